# Initial kernel scaffold; baseline (speedup 1.0000x reference)
import jax
import jax.numpy as jnp
from jax.experimental import pallas as pl


def kernel(input_matrix, adjacency_coo_matrix, W, b):
    raise NotImplementedError("write your pallas kernel here")


# speedup vs baseline: 4.8622x; 4.8622x over previous
import functools

import jax
import jax.numpy as jnp
from jax import lax
from jax.experimental import pallas as pl
from jax.experimental.pallas import tpu as pltpu
from jax.experimental.pallas import tpu_sc as plsc

D = 128
DA = 144
DEG_COL = D
NC, NS = 2, 16
NW = NC * NS
CHUNK = 128


def _sc_aggregate(x_aug, src, dst, n_nodes, e_per_tile):
    n_chunks = e_per_tile // CHUNK
    n_acc = ((n_nodes + 1 + NS * CHUNK - 1) // (NS * CHUNK)) * (NS * CHUNK)
    rows_per_tile = n_acc // NS
    n_pieces = rows_per_tile // CHUNK
    mesh = plsc.VectorSubcoreMesh(core_axis_name="c", subcore_axis_name="s")

    @functools.partial(
        pl.kernel,
        out_type=jax.ShapeDtypeStruct((NC, n_acc, DA), jnp.float32),
        mesh=mesh,
        scratch_types=[
            pltpu.VMEM_SHARED((n_acc, DA), jnp.float32),
            pltpu.VMEM((CHUNK,), jnp.int32),
            pltpu.VMEM((CHUNK,), jnp.int32),
            pltpu.VMEM((CHUNK, DA), jnp.float32),
            pltpu.SemaphoreType.DMA,
        ],
        compiler_params=pltpu.CompilerParams(use_tc_tiling_on_sc=False),
    )
    def agg(x_hbm, src_hbm, dst_hbm, out_hbm, acc, src_v, dst_v, rows_v, sem):
        c = lax.axis_index("c")
        s = lax.axis_index("s")

        zv = jnp.zeros((16,), jnp.float32)

        def fill(i, carry):
            for j in range(DA // 16):
                rows_v[i, pl.ds(j * 16, 16)] = zv
            return carry

        lax.fori_loop(0, CHUNK, fill, 0)
        r0 = s * rows_per_tile
        for k in range(n_pieces):
            pltpu.sync_copy(rows_v, acc.at[pl.ds(r0 + k * CHUNK, CHUNK)])
        plsc.subcore_barrier()

        wid = c * NS + s
        ebase = wid * e_per_tile

        def chunk_body(g, carry):
            b = ebase + g * CHUNK
            pltpu.sync_copy(src_hbm.at[pl.ds(b, CHUNK)], src_v)
            pltpu.sync_copy(dst_hbm.at[pl.ds(b, CHUNK)], dst_v)
            pltpu.async_copy(x_hbm.at[src_v], rows_v, sem).wait()
            pltpu.sync_copy(rows_v, acc.at[dst_v], add=True)
            return carry

        lax.fori_loop(0, n_chunks, chunk_body, 0)
        plsc.subcore_barrier()

        for k in range(n_pieces):
            pltpu.sync_copy(acc.at[pl.ds(r0 + k * CHUNK, CHUNK)], rows_v)
            pltpu.sync_copy(rows_v, out_hbm.at[c, pl.ds(r0 + k * CHUNK, CHUNK)])

    return agg(x_aug, src, dst)


def _tc_head(x, partial, W, b):
    n = x.shape[0]
    R = 1000
    grid = (n // R,)

    def body(x_ref, p_ref, w_ref, b_ref, o_ref):
        xb = x_ref[...]
        p = p_ref[...]
        accb = p[0] + p[1]
        ssum = accb[:, :D]
        deg = accb[:, DEG_COL:DEG_COL + 1]
        mean = ssum / jnp.maximum(deg, 1.0)
        w = w_ref[...]
        h = (
            jnp.dot(xb, w[:D], preferred_element_type=jnp.float32,
                    precision=lax.Precision.HIGHEST)
            + jnp.dot(mean, w[D:], preferred_element_type=jnp.float32,
                      precision=lax.Precision.HIGHEST)
            + b_ref[...]
        )
        h = jnp.maximum(h, 0.0)
        nrm = jnp.sqrt(jnp.sum(h * h, axis=1, keepdims=True))
        o_ref[...] = h / jnp.maximum(nrm, 1e-12)

    return pl.pallas_call(
        body,
        grid=grid,
        in_specs=[
            pl.BlockSpec((R, D), lambda i: (i, 0)),
            pl.BlockSpec((NC, R, DA), lambda i: (0, i, 0)),
            pl.BlockSpec((2 * D, D), lambda i: (0, 0)),
            pl.BlockSpec((1, D), lambda i: (0, 0)),
        ],
        out_specs=pl.BlockSpec((R, D), lambda i: (i, 0)),
        out_shape=jax.ShapeDtypeStruct((n, D), jnp.float32),
    )(x, partial, W, b.reshape(1, D))


def kernel(input_matrix, adjacency_coo_matrix, W, b):
    x = input_matrix
    n = x.shape[0]
    e = adjacency_coo_matrix.shape[1]
    e_per_tile = ((e + NW * CHUNK - 1) // (NW * CHUNK)) * CHUNK
    e_pad = NW * e_per_tile
    pad = e_pad - e
    src = adjacency_coo_matrix[0].astype(jnp.int32)
    dst = adjacency_coo_matrix[1].astype(jnp.int32)
    src_p = jnp.concatenate([src, jnp.zeros((pad,), jnp.int32)])
    dst_p = jnp.concatenate([dst, jnp.full((pad,), n, jnp.int32)])
    ones = jnp.ones((n, 1), x.dtype)
    zpad = jnp.zeros((n, DA - D - 1), x.dtype)
    x_aug = jnp.concatenate([x, ones, zpad], axis=1)
    partial = _sc_aggregate(x_aug, src_p, dst_p, n, e_per_tile)
    return _tc_head(x, partial, W, b)
